# baseline (speedup 1.0000x reference)
import jax
import jax.numpy as jnp
import numpy as np
from jax.experimental import pallas as pl
from jax.experimental.pallas import tpu as pltpu

D = 1152
P = 2
CIN = 4
IMG = 64
GS = IMG // P
N = GS * GS
NEXP = 10
HID = N
MLPH = 4 * D
B = 8
FREQ = 256
OUTC = 8
FD = P * P * OUTC
TN = 1024
NT = N // TN
HC = 512
MC = 2304

_BF = jnp.bfloat16


def _gelu(v):
    return jax.nn.gelu(v, approximate=True)


def _mixer_kernel(ei_ref, x_ref, a_ref, c_ref, gm_ref, b1_ref, b2_ref,
                  ow_ref, ob_ref, os_ref, *wrefs_and_outs, ln_affine):
    wrefs = wrefs_and_outs[:2 * NEXP]
    o_ref = wrefs_and_outs[2 * NEXP]
    w1v, w2v, sem1, sem2 = wrefs_and_outs[2 * NEXP + 1:]
    b = pl.program_id(0)
    slot = jax.lax.rem(b, 2)
    nslot = jax.lax.rem(b + 1, 2)

    def _start(eid, dst_slot):
        for i in range(NEXP):
            @pl.when(eid == i)
            def _(i=i):
                pltpu.make_async_copy(wrefs[i], w1v.at[dst_slot],
                                      sem1.at[dst_slot]).start()
                pltpu.make_async_copy(wrefs[NEXP + i], w2v.at[dst_slot],
                                      sem2.at[dst_slot]).start()

    @pl.when(b == 0)
    def _():
        _start(ei_ref[0], slot)

    @pl.when(b + 1 < B)
    def _():
        _start(ei_ref[jnp.minimum(b + 1, B - 1)], nslot)

    xr = x_ref[0]
    if ln_affine:
        xn = (xr * a_ref[0] + c_ref[0]).astype(_BF)
    else:
        mx = xr * (1.0 + a_ref[0]) + c_ref[0]
        mu = jnp.mean(mx, axis=0, keepdims=True)
        var = jnp.mean(jnp.square(mx - mu), axis=0, keepdims=True)
        xn = ((mx - mu) * jax.lax.rsqrt(var + 1e-5)).astype(_BF)
    pltpu.make_async_copy(wrefs[0], w1v.at[slot], sem1.at[slot]).wait()
    pltpu.make_async_copy(wrefs[NEXP], w2v.at[slot], sem2.at[slot]).wait()
    acc = None
    for j in range(HID // HC):
        sl = slice(j * HC, (j + 1) * HC)
        h = jnp.dot(w1v[slot, sl, :].astype(_BF), xn,
                    preferred_element_type=jnp.float32)
        g = _gelu((h + b1_ref[0, sl, :]).astype(_BF))
        q = jnp.dot(g, ow_ref[...], preferred_element_type=jnp.float32)
        d = jnp.dot(w2v[slot, :, sl].astype(_BF), q.astype(_BF),
                    preferred_element_type=jnp.float32)
        acc = d if acc is None else acc + d
    mo = acc + b2_ref[0] * os_ref[0] + ob_ref[0]
    o_ref[0] = xr + gm_ref[0] * mo


def _mixer_call(ei, xh, a3, c3, gm3, w1l, b1s, w2l, b2s, owT, ob3, os3,
                ln_affine):
    import functools
    hbm = pl.BlockSpec(memory_space=pltpu.MemorySpace.HBM)
    grid_spec = pltpu.PrefetchScalarGridSpec(
        num_scalar_prefetch=1,
        grid=(B,),
        in_specs=[
            pl.BlockSpec((1, N, D), lambda b, ei: (b, 0, 0)),
            pl.BlockSpec((1, 1, D), lambda b, ei: (b, 0, 0)),
            pl.BlockSpec((1, 1, D), lambda b, ei: (b, 0, 0)),
            pl.BlockSpec((1, 1, D), lambda b, ei: (b, 0, 0)),
            pl.BlockSpec((1, HID, 1), lambda b, ei: (ei[b], 0, 0)),
            pl.BlockSpec((1, N, 1), lambda b, ei: (ei[b], 0, 0)),
            pl.BlockSpec((D, D), lambda b, ei: (0, 0)),
            pl.BlockSpec((1, 1, D), lambda b, ei: (0, 0, 0)),
            pl.BlockSpec((1, 1, D), lambda b, ei: (0, 0, 0)),
        ] + [hbm] * (2 * NEXP),
        out_specs=pl.BlockSpec((1, N, D), lambda b, ei: (b, 0, 0)),
        scratch_shapes=[
            pltpu.VMEM((2, HID, N), jnp.float32),
            pltpu.VMEM((2, N, HID), jnp.float32),
            pltpu.SemaphoreType.DMA((2,)),
            pltpu.SemaphoreType.DMA((2,)),
        ],
    )
    return pl.pallas_call(
        functools.partial(_mixer_kernel, ln_affine=ln_affine),
        grid_spec=grid_spec,
        out_shape=jax.ShapeDtypeStruct((B, N, D), jnp.float32),
    )(ei, xh, a3, c3, gm3, b1s, b2s, owT, ob3, os3, *w1l, *w2l)


def _mlp_kernel(x_ref, cl_ref, sl_ref, gl_ref, w1_ref, b1_ref, w2_ref,
                b2_ref, sh_ref, sg_ref, fw_ref, fb_ref, *outs, final):
    xr = x_ref[0]
    mu = jnp.mean(xr, axis=1, keepdims=True)
    var = jnp.mean(jnp.square(xr - mu), axis=1, keepdims=True)
    h = (xr - mu) * jax.lax.rsqrt(var + 1e-6)
    h = (h * (1.0 + cl_ref[0]) + sl_ref[0]).astype(_BF)
    acc = None
    for j in range(MLPH // MC):
        sl = slice(j * MC, (j + 1) * MC)
        a = jnp.dot(h, w1_ref[:, sl], preferred_element_type=jnp.float32)
        g = _gelu((a + b1_ref[0, :, sl]).astype(_BF))
        d = jnp.dot(g, w2_ref[sl, :], preferred_element_type=jnp.float32)
        acc = d if acc is None else acc + d
    val = xr + gl_ref[0] * (acc + b2_ref[0])
    n = pl.program_id(1)
    if final:
        o_ref, = outs
        m2 = jnp.mean(val, axis=1, keepdims=True)
        v2 = jnp.mean(jnp.square(val - m2), axis=1, keepdims=True)
        hf = (val - m2) * jax.lax.rsqrt(v2 + 1e-6)
        hf = (hf * (1.0 + sg_ref[0]) + sh_ref[0]).astype(_BF)
        o_ref[0] = jnp.dot(hf, fw_ref[...],
                           preferred_element_type=jnp.float32) + fb_ref[0]
    else:
        o_ref, s1_ref, s2_ref = outs
        o_ref[0] = val
        p1 = jnp.sum(val, axis=0, keepdims=True)
        p2 = jnp.sum(jnp.square(val), axis=0, keepdims=True)

        @pl.when(n == 0)
        def _():
            s1_ref[0] = p1
            s2_ref[0] = p2

        @pl.when(n > 0)
        def _():
            s1_ref[0] = s1_ref[0] + p1
            s2_ref[0] = s2_ref[0] + p2


def _mlp_call(xh, cl3, sl3, gl3, m1T, mb13, m2T, mb23, sh3, sg3, fwT, fb3,
              final):
    import functools
    if final:
        out_specs = [pl.BlockSpec((1, TN, FD), lambda b, n: (b, n, 0))]
        out_shape = [jax.ShapeDtypeStruct((B, N, FD), jnp.float32)]
    else:
        out_specs = [
            pl.BlockSpec((1, TN, D), lambda b, n: (b, n, 0)),
            pl.BlockSpec((1, 1, D), lambda b, n: (b, 0, 0)),
            pl.BlockSpec((1, 1, D), lambda b, n: (b, 0, 0)),
        ]
        out_shape = [
            jax.ShapeDtypeStruct((B, N, D), jnp.float32),
            jax.ShapeDtypeStruct((B, 1, D), jnp.float32),
            jax.ShapeDtypeStruct((B, 1, D), jnp.float32),
        ]
    return pl.pallas_call(
        functools.partial(_mlp_kernel, final=final),
        grid=(B, NT),
        in_specs=[
            pl.BlockSpec((1, TN, D), lambda b, n: (b, n, 0)),
            pl.BlockSpec((1, 1, D), lambda b, n: (b, 0, 0)),
            pl.BlockSpec((1, 1, D), lambda b, n: (b, 0, 0)),
            pl.BlockSpec((1, 1, D), lambda b, n: (b, 0, 0)),
            pl.BlockSpec((D, MLPH), lambda b, n: (0, 0)),
            pl.BlockSpec((1, 1, MLPH), lambda b, n: (0, 0, 0)),
            pl.BlockSpec((MLPH, D), lambda b, n: (0, 0)),
            pl.BlockSpec((1, 1, D), lambda b, n: (0, 0, 0)),
            pl.BlockSpec((1, 1, D), lambda b, n: (b, 0, 0)),
            pl.BlockSpec((1, 1, D), lambda b, n: (b, 0, 0)),
            pl.BlockSpec((D, FD), lambda b, n: (0, 0)),
            pl.BlockSpec((1, 1, FD), lambda b, n: (0, 0, 0)),
        ],
        out_specs=out_specs,
        out_shape=out_shape,
    )(xh, cl3, sl3, gl3, m1T, mb13, m2T, mb23, sh3, sg3, fwT, fb3)


def _timestep_embedding(t, dim):
    half = dim // 2
    freqs = jnp.exp(-np.log(10000.0) * jnp.arange(half, dtype=jnp.float32) / half)
    args = t[:, None].astype(jnp.float32) * freqs[None]
    return jnp.concatenate([jnp.cos(args), jnp.sin(args)], axis=-1)


def kernel(x, t, y, params):
    p = params
    b = x.shape[0]

    xp = x.reshape(b, CIN, GS, P, GS, P).transpose(0, 2, 4, 1, 3, 5)
    xp = xp.reshape(b, N, CIN * P * P)
    pw = p['patch_w'].reshape(D, CIN * P * P)
    pos = p['pos'][0]
    h = xp @ pw.T + p['patch_b'] + pos

    te = _timestep_embedding(t, FREQ)
    te = jax.nn.silu(te @ p['tw1'].T + p['tb1'])
    te = te @ p['tw2'].T + p['tb2']
    c = te + p['ytab'][y]
    sc = jax.nn.silu(c)

    hmean = jnp.mean(xp, axis=1) @ pw.T + p['patch_b'] + jnp.mean(pos, axis=0)
    hsq = None

    fmod = sc @ p['faw'].T + p['fab']
    fshift, fscale = jnp.split(fmod, 2, axis=1)
    fwT = p['fw'].T.astype(_BF)
    fb3 = p['fb'][None, None, :]
    zero3 = jnp.zeros((B, 1, D), jnp.float32)
    zfd = jnp.zeros((1, 1, FD), jnp.float32)

    aux_total = 0.0
    nblocks = len(p['blocks'])
    for bi, bp in enumerate(p['blocks']):
        mod = sc @ bp['aw'].T + bp['ab']
        sm, cm, gm, sl_, cl, gl = jnp.split(mod, 6, axis=1)
        mom = bp['mom']

        mxm = hmean * (1.0 + cm) + sm
        logits = mxm @ mom['router'].T
        probs = jax.nn.softmax(logits, axis=-1)
        ei = jnp.argmax(probs, axis=-1).astype(jnp.int32)
        em = jax.nn.one_hot(ei, NEXP)
        aux_total = aux_total + NEXP * (probs.mean(0) * em.mean(0)).sum()

        w1l = [e['w1'] for e in mom['experts']]
        w2l = [e['w2'] for e in mom['experts']]
        b1s = jnp.stack([e['b1'] for e in mom['experts']]).reshape(NEXP, HID, 1)
        b2s = jnp.stack([e['b2'] for e in mom['experts']]).reshape(NEXP, N, 1)
        owT = mom['ow'].T.astype(_BF)
        os3 = jnp.sum(mom['ow'], axis=1)[None, None, :]
        ob3 = mom['ob'][None, None, :]

        if hsq is None:
            a3 = cm[:, None, :]
            c3 = sm[:, None, :]
            ln_affine = False
        else:
            var = hsq - jnp.square(hmean)
            s1 = 1.0 + cm
            alpha = s1 * jax.lax.rsqrt(jnp.square(s1) * var + 1e-5)
            a3 = alpha[:, None, :]
            c3 = (-alpha * hmean)[:, None, :]
            ln_affine = True

        x2 = _mixer_call(ei, h, a3, c3, gm[:, None, :], w1l, b1s, w2l, b2s,
                         owT, ob3, os3, ln_affine)

        final = bi == nblocks - 1
        outs = _mlp_call(x2, cl[:, None, :], sl_[:, None, :], gl[:, None, :],
                         bp['m1'].T.astype(_BF), bp['mb1'][None, None, :],
                         bp['m2'].T.astype(_BF), bp['mb2'][None, None, :],
                         fshift[:, None, :] if final else zero3,
                         fscale[:, None, :] if final else zero3,
                         fwT, fb3 if final else zfd, final)
        if final:
            hf = outs[0]
        else:
            h, s1h, s2h = outs
            hmean = s1h[:, 0, :] * (1.0 / N)
            hsq = s2h[:, 0, :] * (1.0 / N)

    hf = hf.reshape(b, GS, GS, P, P, OUTC)
    hf = jnp.einsum('nhwpqc->nchpwq', hf)
    imgs = hf.reshape(b, OUTC, GS * P, GS * P)
    return imgs, 0.01 * aux_total


# speedup vs baseline: 1.0420x; 1.0420x over previous
import jax
import jax.numpy as jnp
import numpy as np
from jax.experimental import pallas as pl
from jax.experimental.pallas import tpu as pltpu

D = 1152
P = 2
CIN = 4
IMG = 64
GS = IMG // P
N = GS * GS
NEXP = 10
HID = N
MLPH = 4 * D
B = 8
FREQ = 256
OUTC = 8
FD = P * P * OUTC
TN = 512
NT = N // TN
HC = 512
MC = 2304

_BF = jnp.bfloat16


def _gelu(v):
    return jax.nn.gelu(v, approximate=True)


def _mixer_kernel(ei_ref, x_ref, a_ref, c_ref, gm_ref, b1_ref, b2_ref,
                  ow_ref, ob_ref, os_ref, *wrefs_and_outs, ln_affine):
    wrefs = wrefs_and_outs[:2 * NEXP]
    o_ref = wrefs_and_outs[2 * NEXP]
    w1v, w2v, sem1, sem2 = wrefs_and_outs[2 * NEXP + 1:]
    b = pl.program_id(0)
    slot = jax.lax.rem(b, 2)
    nslot = jax.lax.rem(b + 1, 2)

    def _start(eid, dst_slot):
        for i in range(NEXP):
            @pl.when(eid == i)
            def _(i=i):
                pltpu.make_async_copy(wrefs[i], w1v.at[dst_slot],
                                      sem1.at[dst_slot]).start()
                pltpu.make_async_copy(wrefs[NEXP + i], w2v.at[dst_slot],
                                      sem2.at[dst_slot]).start()

    @pl.when(b == 0)
    def _():
        _start(ei_ref[0], slot)

    @pl.when(b + 1 < B)
    def _():
        _start(ei_ref[jnp.minimum(b + 1, B - 1)], nslot)

    xr = x_ref[0]
    if ln_affine:
        xn = (xr * a_ref[0] + c_ref[0]).astype(_BF)
    else:
        mx = xr * (1.0 + a_ref[0]) + c_ref[0]
        mu = jnp.mean(mx, axis=0, keepdims=True)
        var = jnp.mean(jnp.square(mx - mu), axis=0, keepdims=True)
        xn = ((mx - mu) * jax.lax.rsqrt(var + 1e-5)).astype(_BF)
    pltpu.make_async_copy(wrefs[0], w1v.at[slot], sem1.at[slot]).wait()
    pltpu.make_async_copy(wrefs[NEXP], w2v.at[slot], sem2.at[slot]).wait()
    acc = None
    for j in range(HID // HC):
        sl = slice(j * HC, (j + 1) * HC)
        h = jnp.dot(w1v[slot, sl, :].astype(_BF), xn,
                    preferred_element_type=jnp.float32)
        g = _gelu((h + b1_ref[0, sl, :]).astype(_BF))
        q = jnp.dot(g, ow_ref[...], preferred_element_type=jnp.float32)
        d = jnp.dot(w2v[slot, :, sl].astype(_BF), q.astype(_BF),
                    preferred_element_type=jnp.float32)
        acc = d if acc is None else acc + d
    mo = acc + b2_ref[0] * os_ref[0] + ob_ref[0]
    o_ref[0] = xr + gm_ref[0] * mo


def _mixer_call(ei, xh, a3, c3, gm3, w1l, b1s, w2l, b2s, owT, ob3, os3,
                ln_affine):
    import functools
    hbm = pl.BlockSpec(memory_space=pltpu.MemorySpace.HBM)
    grid_spec = pltpu.PrefetchScalarGridSpec(
        num_scalar_prefetch=1,
        grid=(B,),
        in_specs=[
            pl.BlockSpec((1, N, D), lambda b, ei: (b, 0, 0)),
            pl.BlockSpec((1, 1, D), lambda b, ei: (b, 0, 0)),
            pl.BlockSpec((1, 1, D), lambda b, ei: (b, 0, 0)),
            pl.BlockSpec((1, 1, D), lambda b, ei: (b, 0, 0)),
            pl.BlockSpec((1, HID, 1), lambda b, ei: (ei[b], 0, 0)),
            pl.BlockSpec((1, N, 1), lambda b, ei: (ei[b], 0, 0)),
            pl.BlockSpec((D, D), lambda b, ei: (0, 0)),
            pl.BlockSpec((1, 1, D), lambda b, ei: (0, 0, 0)),
            pl.BlockSpec((1, 1, D), lambda b, ei: (0, 0, 0)),
        ] + [hbm] * (2 * NEXP),
        out_specs=pl.BlockSpec((1, N, D), lambda b, ei: (b, 0, 0)),
        scratch_shapes=[
            pltpu.VMEM((2, HID, N), jnp.float32),
            pltpu.VMEM((2, N, HID), jnp.float32),
            pltpu.SemaphoreType.DMA((2,)),
            pltpu.SemaphoreType.DMA((2,)),
        ],
    )
    return pl.pallas_call(
        functools.partial(_mixer_kernel, ln_affine=ln_affine),
        grid_spec=grid_spec,
        out_shape=jax.ShapeDtypeStruct((B, N, D), jnp.float32),
    )(ei, xh, a3, c3, gm3, b1s, b2s, owT, ob3, os3, *w1l, *w2l)


def _mlp_kernel(x_ref, cl_ref, sl_ref, gl_ref, w1_ref, b1_ref, w2_ref,
                b2_ref, sh_ref, sg_ref, fw_ref, fb_ref, *outs, final):
    xr = x_ref[0]
    mu = jnp.mean(xr, axis=1, keepdims=True)
    var = jnp.mean(jnp.square(xr - mu), axis=1, keepdims=True)
    h = (xr - mu) * jax.lax.rsqrt(var + 1e-6)
    h = (h * (1.0 + cl_ref[0]) + sl_ref[0]).astype(_BF)
    acc = None
    for j in range(MLPH // MC):
        sl = slice(j * MC, (j + 1) * MC)
        a = jax.lax.dot_general(h, w1_ref[sl, :], (((1,), (1,)), ((), ())),
                                preferred_element_type=jnp.float32)
        g = _gelu((a + b1_ref[0, :, sl]).astype(_BF))
        d = jax.lax.dot_general(g, w2_ref[:, sl], (((1,), (1,)), ((), ())),
                                preferred_element_type=jnp.float32)
        acc = d if acc is None else acc + d
    val = xr + gl_ref[0] * (acc + b2_ref[0])
    n = pl.program_id(1)
    if final:
        o_ref, = outs
        m2 = jnp.mean(val, axis=1, keepdims=True)
        v2 = jnp.mean(jnp.square(val - m2), axis=1, keepdims=True)
        hf = (val - m2) * jax.lax.rsqrt(v2 + 1e-6)
        hf = (hf * (1.0 + sg_ref[0]) + sh_ref[0]).astype(_BF)
        o_ref[0] = jnp.dot(hf, fw_ref[...],
                           preferred_element_type=jnp.float32) + fb_ref[0]
    else:
        o_ref, s1_ref, s2_ref = outs
        o_ref[0] = val
        p1 = jnp.sum(val, axis=0, keepdims=True)
        p2 = jnp.sum(jnp.square(val), axis=0, keepdims=True)

        @pl.when(n == 0)
        def _():
            s1_ref[0] = p1
            s2_ref[0] = p2

        @pl.when(n > 0)
        def _():
            s1_ref[0] = s1_ref[0] + p1
            s2_ref[0] = s2_ref[0] + p2


def _mlp_call(xh, cl3, sl3, gl3, m1T, mb13, m2T, mb23, sh3, sg3, fwT, fb3,
              final):
    import functools
    if final:
        out_specs = [pl.BlockSpec((1, TN, FD), lambda b, n: (b, n, 0))]
        out_shape = [jax.ShapeDtypeStruct((B, N, FD), jnp.float32)]
    else:
        out_specs = [
            pl.BlockSpec((1, TN, D), lambda b, n: (b, n, 0)),
            pl.BlockSpec((1, 1, D), lambda b, n: (b, 0, 0)),
            pl.BlockSpec((1, 1, D), lambda b, n: (b, 0, 0)),
        ]
        out_shape = [
            jax.ShapeDtypeStruct((B, N, D), jnp.float32),
            jax.ShapeDtypeStruct((B, 1, D), jnp.float32),
            jax.ShapeDtypeStruct((B, 1, D), jnp.float32),
        ]
    return pl.pallas_call(
        functools.partial(_mlp_kernel, final=final),
        grid=(B, NT),
        in_specs=[
            pl.BlockSpec((1, TN, D), lambda b, n: (b, n, 0)),
            pl.BlockSpec((1, 1, D), lambda b, n: (b, 0, 0)),
            pl.BlockSpec((1, 1, D), lambda b, n: (b, 0, 0)),
            pl.BlockSpec((1, 1, D), lambda b, n: (b, 0, 0)),
            pl.BlockSpec((MLPH, D), lambda b, n: (0, 0)),
            pl.BlockSpec((1, 1, MLPH), lambda b, n: (0, 0, 0)),
            pl.BlockSpec((D, MLPH), lambda b, n: (0, 0)),
            pl.BlockSpec((1, 1, D), lambda b, n: (0, 0, 0)),
            pl.BlockSpec((1, 1, D), lambda b, n: (b, 0, 0)),
            pl.BlockSpec((1, 1, D), lambda b, n: (b, 0, 0)),
            pl.BlockSpec((D, FD), lambda b, n: (0, 0)),
            pl.BlockSpec((1, 1, FD), lambda b, n: (0, 0, 0)),
        ],
        out_specs=out_specs,
        out_shape=out_shape,
    )(xh, cl3, sl3, gl3, m1T, mb13, m2T, mb23, sh3, sg3, fwT, fb3)


def _timestep_embedding(t, dim):
    half = dim // 2
    freqs = jnp.exp(-np.log(10000.0) * jnp.arange(half, dtype=jnp.float32) / half)
    args = t[:, None].astype(jnp.float32) * freqs[None]
    return jnp.concatenate([jnp.cos(args), jnp.sin(args)], axis=-1)


def kernel(x, t, y, params):
    p = params
    b = x.shape[0]

    xp = x.reshape(b, CIN, GS, P, GS, P).transpose(0, 2, 4, 1, 3, 5)
    xp = xp.reshape(b, N, CIN * P * P)
    pw = p['patch_w'].reshape(D, CIN * P * P)
    pos = p['pos'][0]
    h = xp @ pw.T + p['patch_b'] + pos

    te = _timestep_embedding(t, FREQ)
    te = jax.nn.silu(te @ p['tw1'].T + p['tb1'])
    te = te @ p['tw2'].T + p['tb2']
    c = te + p['ytab'][y]
    sc = jax.nn.silu(c)

    hmean = jnp.mean(xp, axis=1) @ pw.T + p['patch_b'] + jnp.mean(pos, axis=0)
    hsq = None

    fmod = sc @ p['faw'].T + p['fab']
    fshift, fscale = jnp.split(fmod, 2, axis=1)
    fwT = p['fw'].T.astype(_BF)
    fb3 = p['fb'][None, None, :]
    zero3 = jnp.zeros((B, 1, D), jnp.float32)
    zfd = jnp.zeros((1, 1, FD), jnp.float32)

    aux_total = 0.0
    nblocks = len(p['blocks'])
    for bi, bp in enumerate(p['blocks']):
        mod = sc @ bp['aw'].T + bp['ab']
        sm, cm, gm, sl_, cl, gl = jnp.split(mod, 6, axis=1)
        mom = bp['mom']

        mxm = hmean * (1.0 + cm) + sm
        logits = mxm @ mom['router'].T
        probs = jax.nn.softmax(logits, axis=-1)
        ei = jnp.argmax(probs, axis=-1).astype(jnp.int32)
        em = jax.nn.one_hot(ei, NEXP)
        aux_total = aux_total + NEXP * (probs.mean(0) * em.mean(0)).sum()

        w1l = [e['w1'] for e in mom['experts']]
        w2l = [e['w2'] for e in mom['experts']]
        b1s = jnp.stack([e['b1'] for e in mom['experts']]).reshape(NEXP, HID, 1)
        b2s = jnp.stack([e['b2'] for e in mom['experts']]).reshape(NEXP, N, 1)
        owT = mom['ow'].T.astype(_BF)
        os3 = jnp.sum(mom['ow'], axis=1)[None, None, :]
        ob3 = mom['ob'][None, None, :]

        if hsq is None:
            a3 = cm[:, None, :]
            c3 = sm[:, None, :]
            ln_affine = False
        else:
            var = hsq - jnp.square(hmean)
            s1 = 1.0 + cm
            alpha = s1 * jax.lax.rsqrt(jnp.square(s1) * var + 1e-5)
            a3 = alpha[:, None, :]
            c3 = (-alpha * hmean)[:, None, :]
            ln_affine = True

        x2 = _mixer_call(ei, h, a3, c3, gm[:, None, :], w1l, b1s, w2l, b2s,
                         owT, ob3, os3, ln_affine)

        final = bi == nblocks - 1
        outs = _mlp_call(x2, cl[:, None, :], sl_[:, None, :], gl[:, None, :],
                         bp['m1'].astype(_BF), bp['mb1'][None, None, :],
                         bp['m2'].astype(_BF), bp['mb2'][None, None, :],
                         fshift[:, None, :] if final else zero3,
                         fscale[:, None, :] if final else zero3,
                         fwT, fb3 if final else zfd, final)
        if final:
            hf = outs[0]
        else:
            h, s1h, s2h = outs
            hmean = s1h[:, 0, :] * (1.0 / N)
            hsq = s2h[:, 0, :] * (1.0 / N)

    hf = hf.reshape(b, GS, GS, P, P, OUTC)
    hf = jnp.einsum('nhwpqc->nchpwq', hf)
    imgs = hf.reshape(b, OUTC, GS * P, GS * P)
    return imgs, 0.01 * aux_total


# speedup vs baseline: 1.0520x; 1.0096x over previous
import jax
import jax.numpy as jnp
import numpy as np
from jax.experimental import pallas as pl
from jax.experimental.pallas import tpu as pltpu

D = 1152
P = 2
CIN = 4
IMG = 64
GS = IMG // P
N = GS * GS
NEXP = 10
HID = N
MLPH = 4 * D
B = 8
FREQ = 256
OUTC = 8
FD = P * P * OUTC
TN = 512
NT = N // TN
HC = 512
MC = 2304

_BF = jnp.bfloat16


def _gelu(v):
    return jax.nn.gelu(v, approximate=True)


def _mixer_kernel(ei_ref, x_ref, a_ref, c_ref, gm_ref, b1_ref, b2_ref,
                  ow_ref, ob_ref, os_ref, *wrefs_and_outs, ln_affine):
    wrefs = wrefs_and_outs[:2 * NEXP]
    o_ref = wrefs_and_outs[2 * NEXP]
    w1v, w2v, sem1, sem2 = wrefs_and_outs[2 * NEXP + 1:]
    b = pl.program_id(0)
    slot = jax.lax.rem(b, 2)
    nslot = jax.lax.rem(b + 1, 2)

    def _start(eid, dst_slot):
        for i in range(NEXP):
            @pl.when(eid == i)
            def _(i=i):
                pltpu.make_async_copy(wrefs[i], w1v.at[dst_slot],
                                      sem1.at[dst_slot]).start()
                pltpu.make_async_copy(wrefs[NEXP + i], w2v.at[dst_slot],
                                      sem2.at[dst_slot]).start()

    @pl.when(b == 0)
    def _():
        _start(ei_ref[0], slot)

    @pl.when(b + 1 < B)
    def _():
        _start(ei_ref[jnp.minimum(b + 1, B - 1)], nslot)

    xr = x_ref[0]
    if ln_affine:
        xn = (xr * a_ref[0] + c_ref[0]).astype(_BF)
    else:
        mx = xr * (1.0 + a_ref[0]) + c_ref[0]
        mu = jnp.mean(mx, axis=0, keepdims=True)
        var = jnp.mean(jnp.square(mx - mu), axis=0, keepdims=True)
        xn = ((mx - mu) * jax.lax.rsqrt(var + 1e-5)).astype(_BF)
    pltpu.make_async_copy(wrefs[0], w1v.at[slot], sem1.at[slot]).wait()
    qs = []
    for j in range(HID // HC):
        sl = slice(j * HC, (j + 1) * HC)
        h = jnp.dot(w1v[slot, sl, :].astype(_BF), xn,
                    preferred_element_type=jnp.float32)
        g = _gelu((h + b1_ref[0, sl, :]).astype(_BF))
        q = jax.lax.dot_general(g, ow_ref[...], (((1,), (1,)), ((), ())),
                                preferred_element_type=jnp.float32)
        qs.append(q.astype(_BF))
    pltpu.make_async_copy(wrefs[NEXP], w2v.at[slot], sem2.at[slot]).wait()
    acc = jnp.dot(w2v[slot].astype(_BF), jnp.concatenate(qs, axis=0),
                  preferred_element_type=jnp.float32)
    mo = acc + b2_ref[0] * os_ref[0] + ob_ref[0]
    o_ref[0] = xr + gm_ref[0] * mo


def _mixer_call(ei, xh, a3, c3, gm3, w1l, b1s, w2l, b2s, owT, ob3, os3,
                ln_affine):
    import functools
    hbm = pl.BlockSpec(memory_space=pltpu.MemorySpace.HBM)
    grid_spec = pltpu.PrefetchScalarGridSpec(
        num_scalar_prefetch=1,
        grid=(B,),
        in_specs=[
            pl.BlockSpec((1, N, D), lambda b, ei: (b, 0, 0)),
            pl.BlockSpec((1, 1, D), lambda b, ei: (b, 0, 0)),
            pl.BlockSpec((1, 1, D), lambda b, ei: (b, 0, 0)),
            pl.BlockSpec((1, 1, D), lambda b, ei: (b, 0, 0)),
            pl.BlockSpec((1, HID, 1), lambda b, ei: (ei[b], 0, 0)),
            pl.BlockSpec((1, N, 1), lambda b, ei: (ei[b], 0, 0)),
            pl.BlockSpec((D, D), lambda b, ei: (0, 0)),
            pl.BlockSpec((1, 1, D), lambda b, ei: (0, 0, 0)),
            pl.BlockSpec((1, 1, D), lambda b, ei: (0, 0, 0)),
        ] + [hbm] * (2 * NEXP),
        out_specs=pl.BlockSpec((1, N, D), lambda b, ei: (b, 0, 0)),
        scratch_shapes=[
            pltpu.VMEM((2, HID, N), jnp.float32),
            pltpu.VMEM((2, N, HID), jnp.float32),
            pltpu.SemaphoreType.DMA((2,)),
            pltpu.SemaphoreType.DMA((2,)),
        ],
    )
    return pl.pallas_call(
        functools.partial(_mixer_kernel, ln_affine=ln_affine),
        grid_spec=grid_spec,
        out_shape=jax.ShapeDtypeStruct((B, N, D), jnp.float32),
    )(ei, xh, a3, c3, gm3, b1s, b2s, owT, ob3, os3, *w1l, *w2l)


def _mlp_kernel(x_ref, cl_ref, sl_ref, gl_ref, w1_ref, b1_ref, w2_ref,
                b2_ref, sh_ref, sg_ref, fw_ref, fb_ref, *outs, final):
    xr = x_ref[0]
    mu = jnp.mean(xr, axis=1, keepdims=True)
    var = jnp.mean(jnp.square(xr - mu), axis=1, keepdims=True)
    h = (xr - mu) * jax.lax.rsqrt(var + 1e-6)
    h = (h * (1.0 + cl_ref[0]) + sl_ref[0]).astype(_BF)
    acc = None
    for j in range(MLPH // MC):
        sl = slice(j * MC, (j + 1) * MC)
        a = jax.lax.dot_general(h, w1_ref[sl, :], (((1,), (1,)), ((), ())),
                                preferred_element_type=jnp.float32)
        g = _gelu((a + b1_ref[0, :, sl]).astype(_BF))
        d = jax.lax.dot_general(g, w2_ref[:, sl], (((1,), (1,)), ((), ())),
                                preferred_element_type=jnp.float32)
        acc = d if acc is None else acc + d
    val = xr + gl_ref[0] * (acc + b2_ref[0])
    n = pl.program_id(1)
    if final:
        o_ref, = outs
        m2 = jnp.mean(val, axis=1, keepdims=True)
        v2 = jnp.mean(jnp.square(val - m2), axis=1, keepdims=True)
        hf = (val - m2) * jax.lax.rsqrt(v2 + 1e-6)
        hf = (hf * (1.0 + sg_ref[0]) + sh_ref[0]).astype(_BF)
        o_ref[0] = jnp.dot(hf, fw_ref[...],
                           preferred_element_type=jnp.float32) + fb_ref[0]
    else:
        o_ref, s1_ref, s2_ref = outs
        o_ref[0] = val
        p1 = jnp.sum(val, axis=0, keepdims=True)
        p2 = jnp.sum(jnp.square(val), axis=0, keepdims=True)

        @pl.when(n == 0)
        def _():
            s1_ref[0] = p1
            s2_ref[0] = p2

        @pl.when(n > 0)
        def _():
            s1_ref[0] = s1_ref[0] + p1
            s2_ref[0] = s2_ref[0] + p2


def _mlp_call(xh, cl3, sl3, gl3, m1T, mb13, m2T, mb23, sh3, sg3, fwT, fb3,
              final):
    import functools
    if final:
        out_specs = [pl.BlockSpec((1, TN, FD), lambda b, n: (b, n, 0))]
        out_shape = [jax.ShapeDtypeStruct((B, N, FD), jnp.float32)]
    else:
        out_specs = [
            pl.BlockSpec((1, TN, D), lambda b, n: (b, n, 0)),
            pl.BlockSpec((1, 1, D), lambda b, n: (b, 0, 0)),
            pl.BlockSpec((1, 1, D), lambda b, n: (b, 0, 0)),
        ]
        out_shape = [
            jax.ShapeDtypeStruct((B, N, D), jnp.float32),
            jax.ShapeDtypeStruct((B, 1, D), jnp.float32),
            jax.ShapeDtypeStruct((B, 1, D), jnp.float32),
        ]
    return pl.pallas_call(
        functools.partial(_mlp_kernel, final=final),
        grid=(B, NT),
        in_specs=[
            pl.BlockSpec((1, TN, D), lambda b, n: (b, n, 0)),
            pl.BlockSpec((1, 1, D), lambda b, n: (b, 0, 0)),
            pl.BlockSpec((1, 1, D), lambda b, n: (b, 0, 0)),
            pl.BlockSpec((1, 1, D), lambda b, n: (b, 0, 0)),
            pl.BlockSpec((MLPH, D), lambda b, n: (0, 0)),
            pl.BlockSpec((1, 1, MLPH), lambda b, n: (0, 0, 0)),
            pl.BlockSpec((D, MLPH), lambda b, n: (0, 0)),
            pl.BlockSpec((1, 1, D), lambda b, n: (0, 0, 0)),
            pl.BlockSpec((1, 1, D), lambda b, n: (b, 0, 0)),
            pl.BlockSpec((1, 1, D), lambda b, n: (b, 0, 0)),
            pl.BlockSpec((D, FD), lambda b, n: (0, 0)),
            pl.BlockSpec((1, 1, FD), lambda b, n: (0, 0, 0)),
        ],
        out_specs=out_specs,
        out_shape=out_shape,
    )(xh, cl3, sl3, gl3, m1T, mb13, m2T, mb23, sh3, sg3, fwT, fb3)


def _timestep_embedding(t, dim):
    half = dim // 2
    freqs = jnp.exp(-np.log(10000.0) * jnp.arange(half, dtype=jnp.float32) / half)
    args = t[:, None].astype(jnp.float32) * freqs[None]
    return jnp.concatenate([jnp.cos(args), jnp.sin(args)], axis=-1)


def kernel(x, t, y, params):
    p = params
    b = x.shape[0]

    xp = x.reshape(b, CIN, GS, P, GS, P).transpose(0, 2, 4, 1, 3, 5)
    xp = xp.reshape(b, N, CIN * P * P)
    pw = p['patch_w'].reshape(D, CIN * P * P)
    pos = p['pos'][0]
    h = xp @ pw.T + p['patch_b'] + pos

    te = _timestep_embedding(t, FREQ)
    te = jax.nn.silu(te @ p['tw1'].T + p['tb1'])
    te = te @ p['tw2'].T + p['tb2']
    c = te + p['ytab'][y]
    sc = jax.nn.silu(c)

    hmean = jnp.mean(xp, axis=1) @ pw.T + p['patch_b'] + jnp.mean(pos, axis=0)
    hsq = None

    fmod = sc @ p['faw'].T + p['fab']
    fshift, fscale = jnp.split(fmod, 2, axis=1)
    fwT = p['fw'].T.astype(_BF)
    fb3 = p['fb'][None, None, :]
    zero3 = jnp.zeros((B, 1, D), jnp.float32)
    zfd = jnp.zeros((1, 1, FD), jnp.float32)

    aux_total = 0.0
    nblocks = len(p['blocks'])
    for bi, bp in enumerate(p['blocks']):
        mod = sc @ bp['aw'].T + bp['ab']
        sm, cm, gm, sl_, cl, gl = jnp.split(mod, 6, axis=1)
        mom = bp['mom']

        mxm = hmean * (1.0 + cm) + sm
        logits = mxm @ mom['router'].T
        probs = jax.nn.softmax(logits, axis=-1)
        ei = jnp.argmax(probs, axis=-1).astype(jnp.int32)
        em = jax.nn.one_hot(ei, NEXP)
        aux_total = aux_total + NEXP * (probs.mean(0) * em.mean(0)).sum()

        w1l = [e['w1'] for e in mom['experts']]
        w2l = [e['w2'] for e in mom['experts']]
        b1s = jnp.stack([e['b1'] for e in mom['experts']]).reshape(NEXP, HID, 1)
        b2s = jnp.stack([e['b2'] for e in mom['experts']]).reshape(NEXP, N, 1)
        owT = mom['ow'].astype(_BF)
        os3 = jnp.sum(mom['ow'], axis=1)[None, None, :]
        ob3 = mom['ob'][None, None, :]

        if hsq is None:
            a3 = cm[:, None, :]
            c3 = sm[:, None, :]
            ln_affine = False
        else:
            var = hsq - jnp.square(hmean)
            s1 = 1.0 + cm
            alpha = s1 * jax.lax.rsqrt(jnp.square(s1) * var + 1e-5)
            a3 = alpha[:, None, :]
            c3 = (-alpha * hmean)[:, None, :]
            ln_affine = True

        x2 = _mixer_call(ei, h, a3, c3, gm[:, None, :], w1l, b1s, w2l, b2s,
                         owT, ob3, os3, ln_affine)

        final = bi == nblocks - 1
        outs = _mlp_call(x2, cl[:, None, :], sl_[:, None, :], gl[:, None, :],
                         bp['m1'].astype(_BF), bp['mb1'][None, None, :],
                         bp['m2'].astype(_BF), bp['mb2'][None, None, :],
                         fshift[:, None, :] if final else zero3,
                         fscale[:, None, :] if final else zero3,
                         fwT, fb3 if final else zfd, final)
        if final:
            hf = outs[0]
        else:
            h, s1h, s2h = outs
            hmean = s1h[:, 0, :] * (1.0 / N)
            hsq = s2h[:, 0, :] * (1.0 / N)

    hf = hf.reshape(b, GS, GS, P, P, OUTC)
    hf = jnp.einsum('nhwpqc->nchpwq', hf)
    imgs = hf.reshape(b, OUTC, GS * P, GS * P)
    return imgs, 0.01 * aux_total


# speedup vs baseline: 1.0549x; 1.0028x over previous
import jax
import jax.numpy as jnp
import numpy as np
from jax.experimental import pallas as pl
from jax.experimental.pallas import tpu as pltpu

D = 1152
P = 2
CIN = 4
IMG = 64
GS = IMG // P
N = GS * GS
NEXP = 10
HID = N
MLPH = 4 * D
B = 8
FREQ = 256
OUTC = 8
FD = P * P * OUTC
TN = 512
NT = N // TN
HC = 512
MC = 2304

_BF = jnp.bfloat16


def _gelu(v):
    return jax.nn.gelu(v, approximate=True)


def _mixer_kernel(ei_ref, x_ref, a_ref, c_ref, gm_ref, b1_ref, b2_ref,
                  ow_ref, ob_ref, os_ref, *wrefs_and_outs, ln_affine):
    wrefs = wrefs_and_outs[:2 * NEXP]
    o_ref = wrefs_and_outs[2 * NEXP]
    w1v, w2v, sem1, sem2 = wrefs_and_outs[2 * NEXP + 1:]
    b = pl.program_id(0)
    slot = jax.lax.rem(b, 2)
    nslot = jax.lax.rem(b + 1, 2)

    def _start(eid, dst_slot):
        for i in range(NEXP):
            @pl.when(eid == i)
            def _(i=i):
                pltpu.make_async_copy(wrefs[i], w1v.at[dst_slot],
                                      sem1.at[dst_slot]).start()
                pltpu.make_async_copy(wrefs[NEXP + i], w2v.at[dst_slot],
                                      sem2.at[dst_slot]).start()

    @pl.when(b == 0)
    def _():
        _start(ei_ref[0], slot)

    @pl.when(b + 1 < B)
    def _():
        _start(ei_ref[jnp.minimum(b + 1, B - 1)], nslot)

    xr = x_ref[0]
    if ln_affine:
        xn = (xr * a_ref[0] + c_ref[0]).astype(_BF)
    else:
        mx = xr * (1.0 + a_ref[0]) + c_ref[0]
        mu = jnp.mean(mx, axis=0, keepdims=True)
        var = jnp.mean(jnp.square(mx - mu), axis=0, keepdims=True)
        xn = ((mx - mu) * jax.lax.rsqrt(var + 1e-5)).astype(_BF)
    pltpu.make_async_copy(wrefs[0], w1v.at[slot], sem1.at[slot]).wait()
    qs = []
    for j in range(HID // HC):
        sl = slice(j * HC, (j + 1) * HC)
        h = jnp.dot(w1v[slot, sl, :].astype(_BF), xn,
                    preferred_element_type=jnp.float32)
        g = _gelu((h + b1_ref[0, sl, :]).astype(_BF))
        q = jax.lax.dot_general(g, ow_ref[...], (((1,), (1,)), ((), ())),
                                preferred_element_type=jnp.float32)
        qs.append(q.astype(_BF))
    pltpu.make_async_copy(wrefs[NEXP], w2v.at[slot], sem2.at[slot]).wait()
    acc = jnp.dot(w2v[slot].astype(_BF), jnp.concatenate(qs, axis=0),
                  preferred_element_type=jnp.float32)
    mo = acc + b2_ref[0] * os_ref[0] + ob_ref[0]
    o_ref[0] = (xr + gm_ref[0] * mo).astype(_BF)


def _mixer_call(ei, xh, a3, c3, gm3, w1l, b1s, w2l, b2s, owT, ob3, os3,
                ln_affine):
    import functools
    hbm = pl.BlockSpec(memory_space=pltpu.MemorySpace.HBM)
    grid_spec = pltpu.PrefetchScalarGridSpec(
        num_scalar_prefetch=1,
        grid=(B,),
        in_specs=[
            pl.BlockSpec((1, N, D), lambda b, ei: (b, 0, 0)),
            pl.BlockSpec((1, 1, D), lambda b, ei: (b, 0, 0)),
            pl.BlockSpec((1, 1, D), lambda b, ei: (b, 0, 0)),
            pl.BlockSpec((1, 1, D), lambda b, ei: (b, 0, 0)),
            pl.BlockSpec((1, HID, 1), lambda b, ei: (ei[b], 0, 0)),
            pl.BlockSpec((1, N, 1), lambda b, ei: (ei[b], 0, 0)),
            pl.BlockSpec((D, D), lambda b, ei: (0, 0)),
            pl.BlockSpec((1, 1, D), lambda b, ei: (0, 0, 0)),
            pl.BlockSpec((1, 1, D), lambda b, ei: (0, 0, 0)),
        ] + [hbm] * (2 * NEXP),
        out_specs=pl.BlockSpec((1, N, D), lambda b, ei: (b, 0, 0)),
        scratch_shapes=[
            pltpu.VMEM((2, HID, N), jnp.float32),
            pltpu.VMEM((2, N, HID), jnp.float32),
            pltpu.SemaphoreType.DMA((2,)),
            pltpu.SemaphoreType.DMA((2,)),
        ],
    )
    return pl.pallas_call(
        functools.partial(_mixer_kernel, ln_affine=ln_affine),
        grid_spec=grid_spec,
        out_shape=jax.ShapeDtypeStruct((B, N, D), _BF),
    )(ei, xh, a3, c3, gm3, b1s, b2s, owT, ob3, os3, *w1l, *w2l)


def _mlp_kernel(x_ref, cl_ref, sl_ref, gl_ref, w1_ref, b1_ref, w2_ref,
                b2_ref, sh_ref, sg_ref, fw_ref, fb_ref, *outs, final):
    xr = x_ref[0].astype(jnp.float32)
    mu = jnp.mean(xr, axis=1, keepdims=True)
    var = jnp.mean(jnp.square(xr - mu), axis=1, keepdims=True)
    h = (xr - mu) * jax.lax.rsqrt(var + 1e-6)
    h = (h * (1.0 + cl_ref[0]) + sl_ref[0]).astype(_BF)
    acc = None
    for j in range(MLPH // MC):
        sl = slice(j * MC, (j + 1) * MC)
        a = jax.lax.dot_general(h, w1_ref[sl, :], (((1,), (1,)), ((), ())),
                                preferred_element_type=jnp.float32)
        g = _gelu((a + b1_ref[0, :, sl]).astype(_BF))
        d = jax.lax.dot_general(g, w2_ref[:, sl], (((1,), (1,)), ((), ())),
                                preferred_element_type=jnp.float32)
        acc = d if acc is None else acc + d
    val = xr + gl_ref[0] * (acc + b2_ref[0])
    n = pl.program_id(1)
    if final:
        o_ref, = outs
        m2 = jnp.mean(val, axis=1, keepdims=True)
        v2 = jnp.mean(jnp.square(val - m2), axis=1, keepdims=True)
        hf = (val - m2) * jax.lax.rsqrt(v2 + 1e-6)
        hf = (hf * (1.0 + sg_ref[0]) + sh_ref[0]).astype(_BF)
        o_ref[0] = jnp.dot(hf, fw_ref[...],
                           preferred_element_type=jnp.float32) + fb_ref[0]
    else:
        o_ref, s1_ref, s2_ref = outs
        o_ref[0] = val
        p1 = jnp.sum(val, axis=0, keepdims=True)
        p2 = jnp.sum(jnp.square(val), axis=0, keepdims=True)

        @pl.when(n == 0)
        def _():
            s1_ref[0] = p1
            s2_ref[0] = p2

        @pl.when(n > 0)
        def _():
            s1_ref[0] = s1_ref[0] + p1
            s2_ref[0] = s2_ref[0] + p2


def _mlp_call(xh, cl3, sl3, gl3, m1T, mb13, m2T, mb23, sh3, sg3, fwT, fb3,
              final):
    import functools
    if final:
        out_specs = [pl.BlockSpec((1, TN, FD), lambda b, n: (b, n, 0))]
        out_shape = [jax.ShapeDtypeStruct((B, N, FD), jnp.float32)]
    else:
        out_specs = [
            pl.BlockSpec((1, TN, D), lambda b, n: (b, n, 0)),
            pl.BlockSpec((1, 1, D), lambda b, n: (b, 0, 0)),
            pl.BlockSpec((1, 1, D), lambda b, n: (b, 0, 0)),
        ]
        out_shape = [
            jax.ShapeDtypeStruct((B, N, D), jnp.float32),
            jax.ShapeDtypeStruct((B, 1, D), jnp.float32),
            jax.ShapeDtypeStruct((B, 1, D), jnp.float32),
        ]
    return pl.pallas_call(
        functools.partial(_mlp_kernel, final=final),
        grid=(B, NT),
        in_specs=[
            pl.BlockSpec((1, TN, D), lambda b, n: (b, n, 0)),
            pl.BlockSpec((1, 1, D), lambda b, n: (b, 0, 0)),
            pl.BlockSpec((1, 1, D), lambda b, n: (b, 0, 0)),
            pl.BlockSpec((1, 1, D), lambda b, n: (b, 0, 0)),
            pl.BlockSpec((MLPH, D), lambda b, n: (0, 0)),
            pl.BlockSpec((1, 1, MLPH), lambda b, n: (0, 0, 0)),
            pl.BlockSpec((D, MLPH), lambda b, n: (0, 0)),
            pl.BlockSpec((1, 1, D), lambda b, n: (0, 0, 0)),
            pl.BlockSpec((1, 1, D), lambda b, n: (b, 0, 0)),
            pl.BlockSpec((1, 1, D), lambda b, n: (b, 0, 0)),
            pl.BlockSpec((D, FD), lambda b, n: (0, 0)),
            pl.BlockSpec((1, 1, FD), lambda b, n: (0, 0, 0)),
        ],
        out_specs=out_specs,
        out_shape=out_shape,
    )(xh, cl3, sl3, gl3, m1T, mb13, m2T, mb23, sh3, sg3, fwT, fb3)


def _timestep_embedding(t, dim):
    half = dim // 2
    freqs = jnp.exp(-np.log(10000.0) * jnp.arange(half, dtype=jnp.float32) / half)
    args = t[:, None].astype(jnp.float32) * freqs[None]
    return jnp.concatenate([jnp.cos(args), jnp.sin(args)], axis=-1)


def kernel(x, t, y, params):
    p = params
    b = x.shape[0]

    xp = x.reshape(b, CIN, GS, P, GS, P).transpose(0, 2, 4, 1, 3, 5)
    xp = xp.reshape(b, N, CIN * P * P)
    pw = p['patch_w'].reshape(D, CIN * P * P)
    pos = p['pos'][0]
    h = xp @ pw.T + p['patch_b'] + pos

    te = _timestep_embedding(t, FREQ)
    te = jax.nn.silu(te @ p['tw1'].T + p['tb1'])
    te = te @ p['tw2'].T + p['tb2']
    c = te + p['ytab'][y]
    sc = jax.nn.silu(c)

    hmean = jnp.mean(xp, axis=1) @ pw.T + p['patch_b'] + jnp.mean(pos, axis=0)
    hsq = None

    fmod = sc @ p['faw'].T + p['fab']
    fshift, fscale = jnp.split(fmod, 2, axis=1)
    fwT = p['fw'].T.astype(_BF)
    fb3 = p['fb'][None, None, :]
    zero3 = jnp.zeros((B, 1, D), jnp.float32)
    zfd = jnp.zeros((1, 1, FD), jnp.float32)

    aux_total = 0.0
    nblocks = len(p['blocks'])
    for bi, bp in enumerate(p['blocks']):
        mod = sc @ bp['aw'].T + bp['ab']
        sm, cm, gm, sl_, cl, gl = jnp.split(mod, 6, axis=1)
        mom = bp['mom']

        mxm = hmean * (1.0 + cm) + sm
        logits = mxm @ mom['router'].T
        probs = jax.nn.softmax(logits, axis=-1)
        ei = jnp.argmax(probs, axis=-1).astype(jnp.int32)
        em = jax.nn.one_hot(ei, NEXP)
        aux_total = aux_total + NEXP * (probs.mean(0) * em.mean(0)).sum()

        w1l = [e['w1'] for e in mom['experts']]
        w2l = [e['w2'] for e in mom['experts']]
        b1s = jnp.stack([e['b1'] for e in mom['experts']]).reshape(NEXP, HID, 1)
        b2s = jnp.stack([e['b2'] for e in mom['experts']]).reshape(NEXP, N, 1)
        owT = mom['ow'].astype(_BF)
        os3 = jnp.sum(mom['ow'], axis=1)[None, None, :]
        ob3 = mom['ob'][None, None, :]

        if hsq is None:
            a3 = cm[:, None, :]
            c3 = sm[:, None, :]
            ln_affine = False
        else:
            var = hsq - jnp.square(hmean)
            s1 = 1.0 + cm
            alpha = s1 * jax.lax.rsqrt(jnp.square(s1) * var + 1e-5)
            a3 = alpha[:, None, :]
            c3 = (-alpha * hmean)[:, None, :]
            ln_affine = True

        x2 = _mixer_call(ei, h, a3, c3, gm[:, None, :], w1l, b1s, w2l, b2s,
                         owT, ob3, os3, ln_affine)

        final = bi == nblocks - 1
        outs = _mlp_call(x2, cl[:, None, :], sl_[:, None, :], gl[:, None, :],
                         bp['m1'].astype(_BF), bp['mb1'][None, None, :],
                         bp['m2'].astype(_BF), bp['mb2'][None, None, :],
                         fshift[:, None, :] if final else zero3,
                         fscale[:, None, :] if final else zero3,
                         fwT, fb3 if final else zfd, final)
        if final:
            hf = outs[0]
        else:
            h, s1h, s2h = outs
            hmean = s1h[:, 0, :] * (1.0 / N)
            hsq = s2h[:, 0, :] * (1.0 / N)

    hf = hf.reshape(b, GS, GS, P, P, OUTC)
    hf = jnp.einsum('nhwpqc->nchpwq', hf)
    imgs = hf.reshape(b, OUTC, GS * P, GS * P)
    return imgs, 0.01 * aux_total
